# baseline (speedup 1.0000x reference)
import functools

import jax
import jax.numpy as jnp
from jax import lax
from jax.experimental import pallas as pl
from jax.experimental.pallas import tpu as pltpu
from jax.experimental.pallas import tpu_sc as plsc

N_ = 50000
E_ = 400000
D = 128
NS = 16
NC = 2
CBASE = 200
CH = NS * CBASE
EP_ = CH * 128
RNG = 5000
RPC = 5
TRASH = 32
NACCK = RNG + TRASH
WRT = 312
WTAIL = RNG - (NS - 1) * WRT
BM = 1000
NB = N_ // BM


def _seg_body(x, s1, d1, s2, d2, ones_h, zacc_h, zdeg_h,
              agg1, deg1, agg2, deg2,
              gidx, ldst, rows, ones_v, zacc_v, zdeg_v,
              acc, dega, sem):
    c = lax.axis_index("c")
    s = lax.axis_index("s")
    start = s * CBASE

    pltpu.sync_copy(ones_h, ones_v)
    pltpu.sync_copy(zacc_h, zacc_v)
    pltpu.sync_copy(zdeg_h, zdeg_v)

    for rel in range(2):
        src2d = s1 if rel == 0 else s2
        dst2d = d1 if rel == 0 else d2
        agg_o = agg1 if rel == 0 else agg2
        deg_o = deg1 if rel == 0 else deg2

        pltpu.sync_copy(src2d.at[pl.ds(start, CBASE)], gidx)

        for p in range(RPC):
            r0 = (c * RPC + p) * RNG

            pltpu.sync_copy(dst2d.at[pl.ds(start, CBASE)], ldst)

            def _mask(r, _):
                for l in range(8):
                    sl = pl.ds(l * 16, 16)
                    dv = ldst[r, sl]
                    lv = dv - r0
                    ok = (lv >= 0) & (lv < RNG)
                    ldst[r, sl] = jnp.where(ok, lv, RNG + (dv & (TRASH - 1)))
                return 0

            lax.fori_loop(0, CBASE, _mask, 0)

            nz = jnp.where(s < NS - 1, WRT // 8, WTAIL // 8)
            zoff = s * WRT

            def _zero(z, _):
                pltpu.sync_copy(zacc_v, acc.at[pl.ds(zoff + z * 8, 8)])
                pltpu.sync_copy(zdeg_v, dega.at[pl.ds(zoff + z * 8, 8)])
                return 0

            lax.fori_loop(0, nz, _zero, 0)

            plsc.subcore_barrier()

            def _chunk(j, _):
                pltpu.async_copy(x.at[gidx.at[j]], rows, sem).wait()
                pltpu.sync_copy(rows, acc.at[ldst.at[j]], add=True)
                pltpu.sync_copy(ones_v, dega.at[ldst.at[j]], add=True)
                return 0

            lax.fori_loop(0, CBASE, _chunk, 0)

            plsc.subcore_barrier()

            @pl.when(s < NS - 1)
            def _():
                off = pl.multiple_of(s * WRT, 8)
                g = pl.multiple_of(r0 + off, 8)
                pltpu.sync_copy(acc.at[pl.ds(off, WRT)], agg_o.at[pl.ds(g, WRT)])
                pltpu.sync_copy(dega.at[pl.ds(off, WRT)], deg_o.at[pl.ds(g, WRT)])

            @pl.when(s == NS - 1)
            def _():
                off = (NS - 1) * WRT
                g = pl.multiple_of(r0 + off, 8)
                pltpu.sync_copy(acc.at[pl.ds(off, WTAIL)], agg_o.at[pl.ds(g, WTAIL)])
                pltpu.sync_copy(dega.at[pl.ds(off, WTAIL)], deg_o.at[pl.ds(g, WTAIL)])


@functools.lru_cache(maxsize=1)
def _get_seg_kernel():
    mesh = plsc.VectorSubcoreMesh(core_axis_name="c", subcore_axis_name="s")
    return functools.partial(
        pl.kernel,
        out_type=(
            jax.ShapeDtypeStruct((N_, D), jnp.float32),
            jax.ShapeDtypeStruct((N_, 1), jnp.float32),
            jax.ShapeDtypeStruct((N_, D), jnp.float32),
            jax.ShapeDtypeStruct((N_, 1), jnp.float32),
        ),
        mesh=mesh,
        scratch_types=(
            pltpu.VMEM((CBASE, 128), jnp.int32),
            pltpu.VMEM((CBASE, 128), jnp.int32),
            pltpu.VMEM((128, D), jnp.float32),
            pltpu.VMEM((128, 1), jnp.float32),
            pltpu.VMEM((8, D), jnp.float32),
            pltpu.VMEM((8, 1), jnp.float32),
            pltpu.VMEM_SHARED((NACCK, D), jnp.float32),
            pltpu.VMEM_SHARED((NACCK, 1), jnp.float32),
            pltpu.SemaphoreType.DMA,
        ),
    )(_seg_body)


def _seg_kernel(x, s1, d1, s2, d2, *rest):
    def one(s2d, d2d):
        src = s2d.reshape(-1)
        dst = d2d.reshape(-1)
        agg = jax.ops.segment_sum(x[src], dst, num_segments=N_)
        deg = jax.ops.segment_sum(jnp.ones_like(dst, jnp.float32), dst, num_segments=N_)
        return agg, deg.reshape(N_, 1)
    a1, g1 = one(s1, d1)
    a2, g2 = one(s2, d2)
    return a1, g1, a2, g2


def _dense_body(x_ref, a1, d1, a2, d2, wsa, wna, ba, wsb, wnb, bb, o_ref):
    h = x_ref[...]
    hn1 = a1[...] / jnp.maximum(d1[...], 1.0)
    hn2 = a2[...] / jnp.maximum(d2[...], 1.0)
    z1 = (jnp.dot(h, wsa[...], preferred_element_type=jnp.float32)
          + jnp.dot(hn1, wna[...], preferred_element_type=jnp.float32) + ba[...])
    z2 = (jnp.dot(h, wsb[...], preferred_element_type=jnp.float32)
          + jnp.dot(hn2, wnb[...], preferred_element_type=jnp.float32) + bb[...])
    o_ref[...] = 0.5 * (jnp.tanh(z1) + jnp.tanh(z2))


def _dense(x, agg1, deg1, agg2, deg2, wsa, wna, ba, wsb, wnb, bb):
    xspec = pl.BlockSpec((BM, D), lambda i: (i, 0))
    dspec = pl.BlockSpec((BM, 1), lambda i: (i, 0))
    wspec = pl.BlockSpec((D, D), lambda i: (0, 0))
    bspec = pl.BlockSpec((1, D), lambda i: (0, 0))
    return pl.pallas_call(
        _dense_body,
        grid=(NB,),
        in_specs=[xspec, xspec, dspec, xspec, dspec,
                  wspec, wspec, bspec, wspec, wspec, bspec],
        out_specs=pl.BlockSpec((BM, D), lambda i: (i, 0)),
        out_shape=jax.ShapeDtypeStruct((N_, D), jnp.float32),
    )(x, agg1, deg1, agg2, deg2, wsa, wna, ba, wsb, wnb, bb)


def kernel(x, ei_r1_b0, ei_r2_b0, ei_r1_b1, ei_r2_b1,
           Ws1_r1, Wn1_r1, b1_r1, Ws1_r2, Wn1_r2, b1_r2,
           Ws2_r1, Wn2_r1, b2_r1, Ws2_r2, Wn2_r2, b2_r2):
    ones_h = jnp.ones((128, 1), jnp.float32)
    zacc_h = jnp.zeros((8, D), jnp.float32)
    zdeg_h = jnp.zeros((8, 1), jnp.float32)

    npad_e = EP_ - E_
    pad_src = (jnp.arange(npad_e, dtype=jnp.int32) * 97) % N_
    pad_dst = jnp.full((npad_e,), N_, jnp.int32)

    def edges(ei):
        s = jnp.concatenate([ei[0], pad_src]).reshape(CH, 128)
        d = jnp.concatenate([ei[1], pad_dst]).reshape(CH, 128)
        return s, d

    s1, d1 = edges(ei_r1_b0)
    s2, d2 = edges(ei_r2_b0)
    s3, d3 = edges(ei_r1_b1)
    s4, d4 = edges(ei_r2_b1)

    agg1, deg1, agg2, deg2 = _seg_kernel(x, s1, d1, s2, d2, ones_h, zacc_h, zdeg_h)
    h1 = _dense(x, agg1, deg1, agg2, deg2,
                Ws1_r1, Wn1_r1, b1_r1.reshape(1, D), Ws1_r2, Wn1_r2, b1_r2.reshape(1, D))
    agg3, deg3, agg4, deg4 = _seg_kernel(h1, s3, d3, s4, d4, ones_h, zacc_h, zdeg_h)
    return _dense(h1, agg3, deg3, agg4, deg4,
                  Ws2_r1, Wn2_r1, b2_r1.reshape(1, D), Ws2_r2, Wn2_r2, b2_r2.reshape(1, D))


# speedup vs baseline: 1.2689x; 1.2689x over previous
import functools

import jax
import jax.numpy as jnp
from jax import lax
from jax.experimental import pallas as pl
from jax.experimental.pallas import tpu as pltpu
from jax.experimental.pallas import tpu_sc as plsc

N_ = 50000
E_ = 400000
D = 128
NS = 16
NC = 2
CBASE = 200
CH = NS * CBASE
EP_ = CH * 128
RNG = 5000
RPC = 5
TRASH = 32
NACCK = RNG + TRASH
WRT = 312
WTAIL = RNG - (NS - 1) * WRT
BM = 1000
NB = N_ // BM


def _seg_body(x, s1, d1, s2, d2, ones_h, zacc_h, zdeg_h,
              agg1, deg1, agg2, deg2,
              gidx, ldst, rows, ones_v, zacc_v, zdeg_v,
              acc, dega, sem):
    c = lax.axis_index("c")
    s = lax.axis_index("s")
    start = s * CBASE

    pltpu.sync_copy(ones_h, ones_v)
    pltpu.sync_copy(zacc_h, zacc_v)
    pltpu.sync_copy(zdeg_h, zdeg_v)

    for rel in range(2):
        src2d = s1 if rel == 0 else s2
        dst2d = d1 if rel == 0 else d2
        agg_o = agg1 if rel == 0 else agg2
        deg_o = deg1 if rel == 0 else deg2

        pltpu.sync_copy(src2d.at[pl.ds(start, CBASE)], gidx)

        for p in range(RPC):
            r0 = (c * RPC + p) * RNG

            pltpu.sync_copy(dst2d.at[pl.ds(start, CBASE)], ldst)

            def _mask(r, _):
                for l in range(8):
                    sl = pl.ds(l * 16, 16)
                    dv = ldst[r, sl]
                    lv = dv - r0
                    ok = (lv >= 0) & (lv < RNG)
                    ldst[r, sl] = jnp.where(ok, lv, RNG + (dv & (TRASH - 1)))
                return 0

            lax.fori_loop(0, CBASE, _mask, 0)

            nz = jnp.where(s < NS - 1, WRT // 8, WTAIL // 8)
            zoff = s * WRT

            def _zero(z, _):
                pltpu.sync_copy(zacc_v, acc.at[pl.ds(zoff + z * 8, 8)])
                pltpu.sync_copy(zdeg_v, dega.at[pl.ds(zoff + z * 8, 8)])
                return 0

            lax.fori_loop(0, nz, _zero, 0)

            plsc.subcore_barrier()

            def _chunk(j, _):
                pltpu.async_copy(x.at[gidx.at[j]], rows, sem).wait()
                pltpu.sync_copy(rows, acc.at[ldst.at[j]], add=True)
                pltpu.sync_copy(ones_v, dega.at[ldst.at[j]], add=True)
                return 0

            lax.fori_loop(0, CBASE, _chunk, 0)

            plsc.subcore_barrier()

            @pl.when(s < NS - 1)
            def _():
                off = pl.multiple_of(s * WRT, 8)
                g = pl.multiple_of(r0 + off, 8)
                pltpu.sync_copy(acc.at[pl.ds(off, WRT)], agg_o.at[pl.ds(g, WRT)])
                pltpu.sync_copy(dega.at[pl.ds(off, WRT)], deg_o.at[pl.ds(g, WRT)])

            @pl.when(s == NS - 1)
            def _():
                off = (NS - 1) * WRT
                g = pl.multiple_of(r0 + off, 8)
                pltpu.sync_copy(acc.at[pl.ds(off, WTAIL)], agg_o.at[pl.ds(g, WTAIL)])
                pltpu.sync_copy(dega.at[pl.ds(off, WTAIL)], deg_o.at[pl.ds(g, WTAIL)])


@functools.lru_cache(maxsize=1)
def _get_seg_kernel():
    mesh = plsc.VectorSubcoreMesh(core_axis_name="c", subcore_axis_name="s")
    return functools.partial(
        pl.kernel,
        out_type=(
            jax.ShapeDtypeStruct((N_, D), jnp.float32),
            jax.ShapeDtypeStruct((N_, 1), jnp.float32),
            jax.ShapeDtypeStruct((N_, D), jnp.float32),
            jax.ShapeDtypeStruct((N_, 1), jnp.float32),
        ),
        mesh=mesh,
        scratch_types=(
            pltpu.VMEM((CBASE, 128), jnp.int32),
            pltpu.VMEM((CBASE, 128), jnp.int32),
            pltpu.VMEM((128, D), jnp.float32),
            pltpu.VMEM((128, 1), jnp.float32),
            pltpu.VMEM((8, D), jnp.float32),
            pltpu.VMEM((8, 1), jnp.float32),
            pltpu.VMEM_SHARED((NACCK, D), jnp.float32),
            pltpu.VMEM_SHARED((NACCK, 1), jnp.float32),
            pltpu.SemaphoreType.DMA,
        ),
    )(_seg_body)


def _seg_kernel(x, s1, d1, s2, d2, *rest):
    def one(src, dst):
        agg = jax.ops.segment_sum(x[src], dst, num_segments=N_)
        deg = jax.ops.segment_sum(jnp.ones_like(dst, jnp.float32), dst, num_segments=N_)
        return agg, deg.reshape(N_, 1)
    a1, g1 = one(s1, d1)
    a2, g2 = one(s2, d2)
    return a1, g1, a2, g2


def _dense_body(x_ref, a1, d1, a2, d2, wsa, wna, ba, wsb, wnb, bb, o_ref):
    h = x_ref[...]
    hn1 = a1[...] / jnp.maximum(d1[...], 1.0)
    hn2 = a2[...] / jnp.maximum(d2[...], 1.0)
    z1 = (jnp.dot(h, wsa[...], preferred_element_type=jnp.float32)
          + jnp.dot(hn1, wna[...], preferred_element_type=jnp.float32) + ba[...])
    z2 = (jnp.dot(h, wsb[...], preferred_element_type=jnp.float32)
          + jnp.dot(hn2, wnb[...], preferred_element_type=jnp.float32) + bb[...])
    o_ref[...] = 0.5 * (jnp.tanh(z1) + jnp.tanh(z2))


def _dense(x, agg1, deg1, agg2, deg2, wsa, wna, ba, wsb, wnb, bb):
    xspec = pl.BlockSpec((BM, D), lambda i: (i, 0))
    dspec = pl.BlockSpec((BM, 1), lambda i: (i, 0))
    wspec = pl.BlockSpec((D, D), lambda i: (0, 0))
    bspec = pl.BlockSpec((1, D), lambda i: (0, 0))
    return pl.pallas_call(
        _dense_body,
        grid=(NB,),
        in_specs=[xspec, xspec, dspec, xspec, dspec,
                  wspec, wspec, bspec, wspec, wspec, bspec],
        out_specs=pl.BlockSpec((BM, D), lambda i: (i, 0)),
        out_shape=jax.ShapeDtypeStruct((N_, D), jnp.float32),
    )(x, agg1, deg1, agg2, deg2, wsa, wna, ba, wsb, wnb, bb)


def kernel(x, ei_r1_b0, ei_r2_b0, ei_r1_b1, ei_r2_b1,
           Ws1_r1, Wn1_r1, b1_r1, Ws1_r2, Wn1_r2, b1_r2,
           Ws2_r1, Wn2_r1, b2_r1, Ws2_r2, Wn2_r2, b2_r2):
    ones_h = jnp.ones((128, 1), jnp.float32)
    zacc_h = jnp.zeros((8, D), jnp.float32)
    zdeg_h = jnp.zeros((8, 1), jnp.float32)

    def edges(ei):
        return ei[0], ei[1]

    s1, d1 = edges(ei_r1_b0)
    s2, d2 = edges(ei_r2_b0)
    s3, d3 = edges(ei_r1_b1)
    s4, d4 = edges(ei_r2_b1)

    agg1, deg1, agg2, deg2 = _seg_kernel(x, s1, d1, s2, d2, ones_h, zacc_h, zdeg_h)
    h1 = _dense(x, agg1, deg1, agg2, deg2,
                Ws1_r1, Wn1_r1, b1_r1.reshape(1, D), Ws1_r2, Wn1_r2, b1_r2.reshape(1, D))
    agg3, deg3, agg4, deg4 = _seg_kernel(h1, s3, d3, s4, d4, ones_h, zacc_h, zdeg_h)
    return _dense(h1, agg3, deg3, agg4, deg4,
                  Ws2_r1, Wn2_r1, b2_r1.reshape(1, D), Ws2_r2, Wn2_r2, b2_r2.reshape(1, D))
